# baseline (speedup 1.0000x reference)
import functools

import jax
import jax.numpy as jnp
from jax import lax
from jax.experimental import pallas as pl
from jax.experimental.pallas import tpu as pltpu
from jax.experimental.pallas import tpu_sc as plsc

N = 1024
E = 32768
L = 5
EDGE_DIM = 16

HALF = E // 2
NUM_WORKERS = 32
ROWS = 8
CJ = 256
CHUNK = ROWS * CJ
NCHUNK = (N * N) // (CHUNK * NUM_WORKERS)
JQ = N // CJ
TAB_WORDS = L * HALF
LANES = 16
GROUPS = CHUNK // LANES


def _round_bf16_bits(x):
    b = lax.bitcast_convert_type(x, jnp.uint32)
    b = b + jnp.uint32(0x7FFF) + ((b >> jnp.uint32(16)) & jnp.uint32(1))
    return b >> jnp.uint32(16)


def _table_body(at_ref, w_ref, out_ref):
    at = at_ref[...]
    w = w_ref[...]
    s = lax.dot_general(w, at, (((1,), (0,)), ((), ())),
                        preferred_element_type=jnp.float32)
    lo = _round_bf16_bits(s[:, :HALF])
    hi = _round_bf16_bits(s[:, HALF:])
    word = lo | (hi << jnp.uint32(16))
    out_ref[...] = lax.bitcast_convert_type(word, jnp.float32)


_table_call = pl.pallas_call(
    _table_body,
    out_shape=jax.ShapeDtypeStruct((L, HALF), jnp.float32),
)


def _sc_body(idx_hbm, tab_hbm, out_hbm, tab_v, idx_v0, idx_v1, out_v0, out_v1,
             sem_tab, sem_in0, sem_in1, sem_out0, sem_out1):
    wid = lax.axis_index("s") * 2 + lax.axis_index("c")

    idx_bufs = (idx_v0, idx_v1)
    out_bufs = (out_v0, out_v1)
    in_sems = (sem_in0, sem_in1)
    out_sems = (sem_out0, sem_out1)

    tab_cp = pltpu.async_copy(tab_hbm, tab_v, sem_tab)
    tab_l = [tab_v.at[pl.ds(l * HALF, HALF)] for l in range(L)]

    def chunk_coords(k):
        c = wid * NCHUNK + k
        i0 = (c // JQ) * ROWS
        j0 = (c % JQ) * CJ
        return i0, j0

    def start_in(k, b):
        i0, j0 = chunk_coords(k)
        for l in range(L):
            pltpu.async_copy(idx_hbm.at[l, pl.ds(i0, ROWS), pl.ds(j0, CJ)],
                             idx_bufs[b].at[l], in_sems[b])

    def wait_in(b):
        for l in range(L):
            pltpu.make_async_copy(
                idx_hbm.at[0, pl.ds(0, ROWS), pl.ds(0, CJ)],
                idx_bufs[b].at[l], in_sems[b]).wait()

    def wait_out(b):
        pltpu.make_async_copy(
            out_bufs[b], out_hbm.at[pl.ds(0, ROWS), pl.ds(0, CJ)],
            out_sems[b]).wait()

    def compute_chunk(idx_v, out_v):
        def row(r):
            for u in range(CJ // LANES):
                jj = u * LANES
                acc = jnp.zeros((LANES,), jnp.float32)
                for l in range(L):
                    e = idx_v[l, r, pl.ds(jj, LANES)]
                    word_ix = lax.bitwise_and(e, jnp.int32(HALF - 1))
                    w = plsc.load_gather(tab_l[l], [word_ix])
                    raw = plsc.bitcast(w, jnp.int32)
                    lo = plsc.bitcast(lax.shift_left(raw, 16), jnp.float32)
                    hi = plsc.bitcast(lax.bitwise_and(raw, jnp.int32(-65536)),
                                      jnp.float32)
                    v = jnp.where(e < HALF, lo, hi)
                    acc = acc + v
                out_v[r, pl.ds(jj, LANES)] = acc * jnp.float32(1.0 / L)

        plsc.parallel_loop(0, ROWS)(row)

    start_in(0, 0)
    start_in(1, 1)
    tab_cp.wait()

    def chunk_pair(kp, carry):
        for b in range(2):
            k = kp * 2 + b
            wait_in(b)

            @pl.when(k >= 2)
            def _():
                wait_out(b)

            compute_chunk(idx_bufs[b], out_bufs[b])
            i0, j0 = chunk_coords(k)
            pltpu.async_copy(out_bufs[b],
                             out_hbm.at[pl.ds(i0, ROWS), pl.ds(j0, CJ)],
                             out_sems[b])

            @pl.when(k + 2 < NCHUNK)
            def _():
                start_in(k + 2, b)
        return carry

    lax.fori_loop(0, NCHUNK // 2, chunk_pair, 0)
    wait_out(0)
    wait_out(1)


_sc_call = functools.partial(
    pl.kernel,
    out_type=jax.ShapeDtypeStruct((N, N), jnp.float32),
    mesh=plsc.VectorSubcoreMesh(core_axis_name="c", subcore_axis_name="s"),
    compiler_params=pltpu.CompilerParams(needs_layout_passes=False),
    scratch_types=[
        pltpu.VMEM((TAB_WORDS,), jnp.float32),
        pltpu.VMEM((L, ROWS, CJ), jnp.int32),
        pltpu.VMEM((L, ROWS, CJ), jnp.int32),
        pltpu.VMEM((ROWS, CJ), jnp.float32),
        pltpu.VMEM((ROWS, CJ), jnp.float32),
        pltpu.SemaphoreType.DMA,
        pltpu.SemaphoreType.DMA,
        pltpu.SemaphoreType.DMA,
        pltpu.SemaphoreType.DMA,
        pltpu.SemaphoreType.DMA,
    ],
)(_sc_body)


def kernel(x, edge_attr, edge_paths, edge_weights):
    del x
    idx = jnp.transpose(edge_paths.astype(jnp.int32), (2, 0, 1))
    words = _table_call(edge_attr.T, edge_weights).reshape(-1)
    return _sc_call(idx, words)


# speedup vs baseline: 1.1505x; 1.1505x over previous
import functools

import jax
import jax.numpy as jnp
from jax import lax
from jax.experimental import pallas as pl
from jax.experimental.pallas import tpu as pltpu
from jax.experimental.pallas import tpu_sc as plsc

N = 1024
E = 32768
L = 5
EDGE_DIM = 16

HALF = E // 2
NUM_WORKERS = 32
ROWS = 8
CJ = 256
CHUNK = ROWS * CJ
NCHUNK = (N * N) // (CHUNK * NUM_WORKERS)
JQ = N // CJ
TAB_WORDS = L * HALF
LANES = 16
GROUPS = CHUNK // LANES


def _round_bf16_bits(x):
    b = lax.bitcast_convert_type(x, jnp.uint32)
    b = b + jnp.uint32(0x7FFF) + ((b >> jnp.uint32(16)) & jnp.uint32(1))
    return b >> jnp.uint32(16)


def _table_body(at_ref, w_ref, out_ref):
    at = at_ref[...]
    w = w_ref[...]
    s = lax.dot_general(w, at, (((1,), (0,)), ((), ())),
                        preferred_element_type=jnp.float32)
    lo = _round_bf16_bits(s[:, :HALF])
    hi = _round_bf16_bits(s[:, HALF:])
    word = lo | (hi << jnp.uint32(16))
    out_ref[...] = lax.bitcast_convert_type(word, jnp.float32)


_table_call = pl.pallas_call(
    _table_body,
    out_shape=jax.ShapeDtypeStruct((L, HALF), jnp.float32),
)


def _sc_body(idx_hbm, tab_hbm, out_hbm, tab_v, idx_v0, idx_v1, out_v0, out_v1,
             sem_tab, sem_in0, sem_in1, sem_out0, sem_out1):
    wid = lax.axis_index("s") * 2 + lax.axis_index("c")

    idx_bufs = (idx_v0, idx_v1)
    out_bufs = (out_v0, out_v1)
    in_sems = (sem_in0, sem_in1)
    out_sems = (sem_out0, sem_out1)

    tab_cp = pltpu.async_copy(tab_hbm, tab_v, sem_tab)
    tab_l = [tab_v.at[pl.ds(l * HALF, HALF)] for l in range(L)]

    def chunk_coords(k):
        c = wid * NCHUNK + k
        i0 = (c // JQ) * ROWS
        j0 = (c % JQ) * CJ
        return i0, j0

    in_cps = {}

    def start_in(k):
        i0, j0 = chunk_coords(k)
        b = k & 1
        in_cps[k] = [
            pltpu.async_copy(idx_hbm.at[l, pl.ds(i0, ROWS), pl.ds(j0, CJ)],
                             idx_bufs[b].at[l], in_sems[b])
            for l in range(L)
        ]

    start_in(0)
    start_in(1)
    tab_cp.wait()

    out_cps = {}
    for k in range(NCHUNK):
        b = k & 1
        for cp in in_cps[k]:
            cp.wait()
        if k >= 2:
            out_cps[k - 2].wait()
        idx_v = idx_bufs[b]
        out_v = out_bufs[b]

        def group(g, idx_v=idx_v, out_v=out_v):
            r = g >> 4
            jj = (g & 15) * LANES
            acc = jnp.zeros((LANES,), jnp.float32)
            for l in range(L):
                e = idx_v[l, r, pl.ds(jj, LANES)]
                word_ix = lax.bitwise_and(e, jnp.int32(HALF - 1))
                w = plsc.load_gather(tab_l[l], [word_ix])
                raw = plsc.bitcast(w, jnp.int32)
                lo = plsc.bitcast(lax.shift_left(raw, 16), jnp.float32)
                hi = plsc.bitcast(lax.bitwise_and(raw, jnp.int32(-65536)),
                                  jnp.float32)
                v = jnp.where(e < HALF, lo, hi)
                acc = acc + v
            out_v[r, pl.ds(jj, LANES)] = acc * jnp.float32(1.0 / L)

        plsc.parallel_loop(0, GROUPS, unroll=4)(group)

        i0, j0 = chunk_coords(k)
        out_cps[k] = pltpu.async_copy(
            out_v, out_hbm.at[pl.ds(i0, ROWS), pl.ds(j0, CJ)], out_sems[b])
        if k + 2 < NCHUNK:
            start_in(k + 2)

    out_cps[NCHUNK - 2].wait()
    out_cps[NCHUNK - 1].wait()


_sc_call = functools.partial(
    pl.kernel,
    out_type=jax.ShapeDtypeStruct((N, N), jnp.float32),
    mesh=plsc.VectorSubcoreMesh(core_axis_name="c", subcore_axis_name="s"),
    compiler_params=pltpu.CompilerParams(needs_layout_passes=False),
    scratch_types=[
        pltpu.VMEM((TAB_WORDS,), jnp.float32),
        pltpu.VMEM((L, ROWS, CJ), jnp.int32),
        pltpu.VMEM((L, ROWS, CJ), jnp.int32),
        pltpu.VMEM((ROWS, CJ), jnp.float32),
        pltpu.VMEM((ROWS, CJ), jnp.float32),
        pltpu.SemaphoreType.DMA,
        pltpu.SemaphoreType.DMA,
        pltpu.SemaphoreType.DMA,
        pltpu.SemaphoreType.DMA,
        pltpu.SemaphoreType.DMA,
    ],
)(_sc_body)


def kernel(x, edge_attr, edge_paths, edge_weights):
    del x
    idx = jnp.transpose(edge_paths.astype(jnp.int32), (2, 0, 1))
    words = _table_call(edge_attr.T, edge_weights).reshape(-1)
    return _sc_call(idx, words)
